# baseline (speedup 1.0000x reference)
import functools

import jax
import jax.numpy as jnp
from jax import lax
from jax.experimental import pallas as pl
from jax.experimental.pallas import tpu as pltpu
from jax.experimental.pallas import tpu_sc as plsc

_B = 1024
_D = 32
_HID = 128
_TILE_V = 2048


def _sc_gather4(item_id, user_segment, region, device_type,
                E_item, E_seg, E_region, E_device):
  info = plsc.get_sparse_core_info()
  nc, ns = info.num_cores, info.num_subcores
  nw = nc * ns
  bpw = _B // nw

  mesh = plsc.VectorSubcoreMesh(core_axis_name="c", subcore_axis_name="s")
  out_t = [jax.ShapeDtypeStruct((_B, _D), jnp.float32) for _ in range(4)]

  @functools.partial(
      pl.kernel,
      out_type=out_t,
      mesh=mesh,
      compiler_params=pltpu.CompilerParams(use_tc_tiling_on_sc=False),
      scratch_types=[
          pltpu.VMEM((4, bpw), jnp.int32),
          pltpu.VMEM((4, bpw, _D), jnp.float32),
          pltpu.SemaphoreType.DMA,
          pltpu.SemaphoreType.DMA,
          pltpu.SemaphoreType.DMA,
          pltpu.SemaphoreType.DMA,
      ],
  )
  def k(idx0, idx1, idx2, idx3, t0, t1, t2, t3,
        o0, o1, o2, o3, idx_v, rows_v, s0, s1, s2, s3):
    wid = lax.axis_index("s") * nc + lax.axis_index("c")
    base = wid * bpw
    idxs = (idx0, idx1, idx2, idx3)
    tabs = (t0, t1, t2, t3)
    outs = (o0, o1, o2, o3)
    sems = (s0, s1, s2, s3)
    for v in range(4):
      pltpu.sync_copy(idxs[v].at[pl.ds(base, bpw)], idx_v.at[v])
    descs = [
        pltpu.async_copy(tabs[v].at[idx_v.at[v]], rows_v.at[v], sems[v])
        for v in range(4)
    ]
    for v in range(4):
      descs[v].wait()
      pltpu.sync_copy(rows_v.at[v], outs[v].at[pl.ds(base, bpw)])

  return k(item_id, user_segment, region, device_type,
           E_item, E_seg, E_region, E_device)


def _dense_body(e_i, e_s, e_r, e_d, w1, b1, w2, b2,
                hwi, hbi, hws, hbs, hwr, hbr, hwd, hbd,
                emb_out, li_out, ls_out, lr_out, ld_out):
  step = pl.program_id(0)

  @pl.when(step == 0)
  def _():
    x = jnp.dot(e_i[...], w1[0 * _D:1 * _D, :], preferred_element_type=jnp.float32)
    x += jnp.dot(e_s[...], w1[1 * _D:2 * _D, :], preferred_element_type=jnp.float32)
    x += jnp.dot(e_r[...], w1[2 * _D:3 * _D, :], preferred_element_type=jnp.float32)
    x += jnp.dot(e_d[...], w1[3 * _D:4 * _D, :], preferred_element_type=jnp.float32)
    h = jnp.maximum(x + b1[...], 0.0)
    emb = jnp.dot(h, w2[...], preferred_element_type=jnp.float32) + b2[...]
    emb_out[...] = emb
    ls_out[...] = jnp.dot(emb, hws[...], preferred_element_type=jnp.float32) + hbs[...]
    lr_out[...] = jnp.dot(emb, hwr[...], preferred_element_type=jnp.float32) + hbr[...]
    ld_out[...] = jnp.dot(emb, hwd[...], preferred_element_type=jnp.float32) + hbd[...]

  emb = emb_out[...]
  li_out[...] = jnp.dot(emb, hwi[...], preferred_element_type=jnp.float32) + hbi[...]


def kernel(item_id, user_segment, region, device_type,
           E_item, E_seg, E_region, E_device,
           W1, b1, W2, b2,
           Hw_item, Hb_item, Hw_seg, Hb_seg,
           Hw_region, Hb_region, Hw_device, Hb_device):
  e_item, e_seg, e_region, e_device = _sc_gather4(
      item_id, user_segment, region, device_type,
      E_item, E_seg, E_region, E_device)

  v_item = Hw_item.shape[1]
  v_seg = Hw_seg.shape[1]
  v_region = Hw_region.shape[1]
  v_device = Hw_device.shape[1]
  n_tiles = pl.cdiv(v_item, _TILE_V)

  const2 = lambda i: (0, 0)
  full = lambda shape: pl.BlockSpec(shape, const2)

  grid_spec = pl.GridSpec(
      grid=(n_tiles,),
      in_specs=[
          full((_B, _D)), full((_B, _D)), full((_B, _D)), full((_B, _D)),
          full((4 * _D, _HID)), full((1, _HID)),
          full((_HID, _D)), full((1, _D)),
          pl.BlockSpec((_D, _TILE_V), lambda i: (0, i)),
          pl.BlockSpec((1, _TILE_V), lambda i: (0, i)),
          full((_D, v_seg)), full((1, v_seg)),
          full((_D, v_region)), full((1, v_region)),
          full((_D, v_device)), full((1, v_device)),
      ],
      out_specs=[
          full((_B, _D)),
          pl.BlockSpec((_B, _TILE_V), lambda i: (0, i)),
          full((_B, v_seg)),
          full((_B, v_region)),
          full((_B, v_device)),
      ],
  )

  out_shape = [
      jax.ShapeDtypeStruct((_B, _D), jnp.float32),
      jax.ShapeDtypeStruct((_B, v_item), jnp.float32),
      jax.ShapeDtypeStruct((_B, v_seg), jnp.float32),
      jax.ShapeDtypeStruct((_B, v_region), jnp.float32),
      jax.ShapeDtypeStruct((_B, v_device), jnp.float32),
  ]

  emb, li, ls, lr, ld = pl.pallas_call(
      _dense_body,
      grid_spec=grid_spec,
      out_shape=out_shape,
  )(e_item, e_seg, e_region, e_device,
    W1, b1.reshape(1, _HID), W2, b2.reshape(1, _D),
    Hw_item, Hb_item.reshape(1, v_item),
    Hw_seg, Hb_seg.reshape(1, v_seg),
    Hw_region, Hb_region.reshape(1, v_region),
    Hw_device, Hb_device.reshape(1, v_device))

  return (emb, li, ls, lr, ld)
